# Initial kernel scaffold; baseline (speedup 1.0000x reference)
import jax
import jax.numpy as jnp
from jax.experimental import pallas as pl


def kernel(pos, batch, W1, b1, g1, be1, W2, b2, g2, be2, W3, b3, g3, be3, W4, b4, g4, be4, W5, b5, g5, be5, W6, b6, g6, be6, W7, b7):
    raise NotImplementedError("write your pallas kernel here")


# speedup vs baseline: 6.3550x; 6.3550x over previous
import jax
import jax.numpy as jnp
from jax.experimental import pallas as pl

_K = 20
_BIG = 1e30
_HI = jax.lax.Precision.HIGHEST


def _hmm(a, b):
    return jax.lax.dot_general(a, b, (((1,), (0,)), ((), ())), precision=_HI)


def _bmm(a, b):
    return jax.lax.dot_general(a.astype(jnp.bfloat16),
                               b.astype(jnp.bfloat16),
                               (((1,), (0,)), ((), ())),
                               preferred_element_type=jnp.float32)


def _edge_body(x_ref, sc_ref, cu_ref, ch_ref, w_ref, b_ref,
               zmx_out, zmn_out, sz_out, sz2_out):
    x = x_ref[0]
    P = x.shape[0]
    dout = w_ref.shape[1]
    ssq = sc_ref[0]
    xh = x.astype(jnp.bfloat16)
    d2 = jax.lax.dot_general(xh, xh, (((1,), (1,)), ((), ())),
                             preferred_element_type=jnp.float32)
    ones = jnp.ones((P, 1), jnp.float32)
    srow = jax.lax.dot_general(ones, ssq, (((1,), (1,)), ((), ())),
                               precision=_HI)
    dist = (ssq + srow) - 2.0 * d2
    pa = x.astype(jnp.bfloat16)
    r1 = x - pa.astype(jnp.float32)
    pb = r1.astype(jnp.bfloat16)
    pc = (r1 - pb.astype(jnp.float32)).astype(jnp.bfloat16)
    cu16 = cu_ref[...].astype(jnp.bfloat16)
    ch16 = ch_ref[...].astype(jnp.bfloat16)

    def bdot(p, q):
        return jax.lax.dot_general(p, q, (((1,), (0,)), ((), ())),
                                   preferred_element_type=jnp.float32)

    u = (bdot(pa, cu16) + bdot(pb, cu16)) + bdot(pc, cu16)
    pha = bdot(pa, ch16).astype(jnp.bfloat16)
    phb = bdot(pb, ch16).astype(jnp.bfloat16)
    phc = bdot(pc, ch16).astype(jnp.bfloat16)
    bias = b_ref[...]
    wb16 = w_ref[...].astype(jnp.bfloat16)
    lane = jax.lax.broadcasted_iota(jnp.int32, (P, P), 1)

    def step(_, carry):
        dc, s1, s2, mx, mn = carry
        rmin = jnp.min(dc, axis=1, keepdims=True)
        cand = dc == rmin
        idx = jnp.min(jnp.where(cand, lane, P), axis=1, keepdims=True)
        onehot = lane == idx
        oh16 = onehot.astype(jnp.bfloat16)
        fk = u + ((bdot(oh16, pha) + bdot(oh16, phb)) + bdot(oh16, phc))
        zk = jax.lax.dot_general(fk.astype(jnp.bfloat16), wb16,
                                 (((1,), (0,)), ((), ())),
                                 preferred_element_type=jnp.float32) + bias
        return (jnp.where(onehot, _BIG, dc), s1 + zk, s2 + zk * zk,
                jnp.maximum(mx, zk), jnp.minimum(mn, zk))

    zeros = jnp.zeros((P, dout), jnp.float32)
    dc, s1, s2, mx, mn = jax.lax.fori_loop(
        0, _K, step,
        (dist, zeros, zeros, jnp.full((P, dout), -_BIG, jnp.float32),
         jnp.full((P, dout), _BIG, jnp.float32)))
    zmx_out[0] = mx
    zmn_out[0] = mn
    sz_out[0] = jnp.sum(s1, axis=0, keepdims=True)
    sz2_out[0] = jnp.sum(s2, axis=0, keepdims=True)


def _edge(x, scol, cu, ch, w, bias):
    B, P, d = x.shape
    f = cu.shape[1]
    dout = w.shape[1]
    return pl.pallas_call(
        _edge_body,
        grid=(B,),
        in_specs=[pl.BlockSpec((1, P, d), lambda b: (b, 0, 0)),
                  pl.BlockSpec((1, P, 1), lambda b: (b, 0, 0)),
                  pl.BlockSpec((d, f), lambda b: (0, 0)),
                  pl.BlockSpec((d, f), lambda b: (0, 0)),
                  pl.BlockSpec((f, dout), lambda b: (0, 0)),
                  pl.BlockSpec((1, dout), lambda b: (0, 0))],
        out_specs=[pl.BlockSpec((1, P, dout), lambda b: (b, 0, 0)),
                   pl.BlockSpec((1, P, dout), lambda b: (b, 0, 0)),
                   pl.BlockSpec((1, 1, dout), lambda b: (b, 0, 0)),
                   pl.BlockSpec((1, 1, dout), lambda b: (b, 0, 0))],
        out_shape=[jax.ShapeDtypeStruct((B, P, dout), jnp.float32),
                   jax.ShapeDtypeStruct((B, P, dout), jnp.float32),
                   jax.ShapeDtypeStruct((B, 1, dout), jnp.float32),
                   jax.ShapeDtypeStruct((B, 1, dout), jnp.float32)],
    )(x, scol, cu, ch, w, bias)


def _feat_body(x1_ref, x2_ref, x3_ref, p1_ref, p2_ref, p3_ref, w4_ref,
               b4_ref, ymx_out, ymn_out, sy_out, sy2_out):
    xcat = (_hmm(x1_ref[0], p1_ref[...]) + _hmm(x2_ref[0], p2_ref[...])
            + _hmm(x3_ref[0], p3_ref[...]))
    y = _bmm(xcat, w4_ref[...]) + b4_ref[...]
    ymx_out[0] = jnp.max(y, axis=0, keepdims=True)
    ymn_out[0] = jnp.min(y, axis=0, keepdims=True)
    sy_out[0] = jnp.sum(y, axis=0, keepdims=True)
    sy2_out[0] = jnp.sum(y * y, axis=0, keepdims=True)


def _feat(x1, x2, x3, p1, p2, p3, w4, b4):
    B, P, d1 = x1.shape
    d2 = x2.shape[2]
    d3 = x3.shape[2]
    H = w4.shape[1]
    F = w4.shape[0]
    in_specs = [
        pl.BlockSpec((1, P, d1), lambda b: (b, 0, 0)),
        pl.BlockSpec((1, P, d2), lambda b: (b, 0, 0)),
        pl.BlockSpec((1, P, d3), lambda b: (b, 0, 0)),
        pl.BlockSpec((d1, F), lambda b: (0, 0)),
        pl.BlockSpec((d2, F), lambda b: (0, 0)),
        pl.BlockSpec((d3, F), lambda b: (0, 0)),
        pl.BlockSpec((F, H), lambda b: (0, 0)),
        pl.BlockSpec((1, H), lambda b: (0, 0)),
    ]
    out_specs = [pl.BlockSpec((1, 1, H), lambda b: (b, 0, 0))
                 for _ in range(4)]
    out_shape = [jax.ShapeDtypeStruct((B, 1, H), jnp.float32)
                 for _ in range(4)]
    return pl.pallas_call(
        _feat_body, grid=(B,), in_specs=in_specs,
        out_specs=out_specs, out_shape=out_shape,
    )(x1, x2, x3, p1, p2, p3, w4, b4)


def _head_body(ymx, ymn, st4, w5, b5, g5, be5, w6, b6, g6, be6, w7, b7,
               out):
    m4 = st4[0:1, :]
    v4 = st4[1:2, :]
    g4 = st4[2:3, :]
    be4 = st4[3:4, :]
    ysel = jnp.where(g4 >= 0.0, ymx[...], ymn[...])
    pooled = jnp.maximum((ysel - m4) / jnp.sqrt(v4 + 1e-5) * g4 + be4, 0.0)
    h = _bmm(pooled, w5[...]) + b5[...]
    m = jnp.mean(h, axis=0, keepdims=True)
    v = jnp.mean((h - m) * (h - m), axis=0, keepdims=True)
    h = jnp.maximum((h - m) / jnp.sqrt(v + 1e-5) * g5[...] + be5[...], 0.0)
    h = _bmm(h, w6[...]) + b6[...]
    m = jnp.mean(h, axis=0, keepdims=True)
    v = jnp.mean((h - m) * (h - m), axis=0, keepdims=True)
    h = jnp.maximum((h - m) / jnp.sqrt(v + 1e-5) * g6[...] + be6[...], 0.0)
    out[...] = _bmm(h, w7[...]) + b7[...]


def _head(ymx, ymn, st4, w5, b5, g5, be5, w6, b6, g6, be6, w7, b7):
    B = ymx.shape[0]
    D7 = w7.shape[1]
    return pl.pallas_call(
        _head_body,
        out_shape=jax.ShapeDtypeStruct((B, D7), jnp.float32),
    )(ymx, ymn, st4, w5, b5, g5, be5, w6, b6, g6, be6, w7, b7)


def _edge_glue(B, P, zmx, zmn, sz, sz2, g, be):
    nk = jnp.float32(B * P * _K)
    dout = zmx.shape[2]
    m = jnp.sum(sz[:, 0, :], axis=0) / nk
    v = jnp.sum(sz2[:, 0, :], axis=0) / nk - m * m
    zsel = jnp.where(g >= 0.0, zmx, zmn).reshape(B * P, dout)
    x = jax.nn.relu((zsel - m) / jnp.sqrt(v + 1e-5) * g + be)
    xr = x.reshape(B, P, dout)
    s = jnp.sum(xr * xr, axis=-1)
    return xr, s.reshape(B, P, 1)


def _forward(pos, W1, b1, g1, be1, W2, b2, g2, be2, W3, b3, g3, be3,
             W4, b4, g4, be4, W5, b5, g5, be5, W6, b6, g6, be6, W7, b7, B):
    N = pos.shape[0]
    P = N // B
    f32 = jnp.float32

    x0 = jnp.pad(pos, ((0, 0), (0, 5))).reshape(B, P, 8)
    xr0 = pos.reshape(B, P, 3)
    s0 = jnp.sum(xr0 * xr0, axis=-1).reshape(B, P, 1)
    i3 = jnp.arange(3)
    cu8 = jnp.zeros((8, 8), f32).at[i3, i3].set(1.0).at[i3, i3 + 3].set(-1.0)
    ch8 = jnp.zeros((8, 8), f32).at[i3, i3 + 3].set(1.0)
    w8 = jnp.pad(W1, ((0, 2), (0, 0)))

    zmx1, zmn1, sz1, sz21 = _edge(x0, s0, cu8, ch8, w8, b1[None])
    x1, s1c = _edge_glue(B, P, zmx1, zmn1, sz1, sz21, g1, be1)

    eye = jnp.eye(64, dtype=f32)
    zz = jnp.zeros((64, 64), f32)
    cu64 = jnp.concatenate([eye, -eye], axis=1)
    ch64 = jnp.concatenate([zz, eye], axis=1)

    zmx2, zmn2, sz2, sz22 = _edge(x1, s1c, cu64, ch64, W2, b2[None])
    x2, s2c = _edge_glue(B, P, zmx2, zmn2, sz2, sz22, g2, be2)

    zmx3, zmn3, sz3, sz23 = _edge(x2, s2c, cu64, ch64, W3, b3[None])
    x3, _ = _edge_glue(B, P, zmx3, zmn3, sz3, sz23, g3, be3)

    p1 = jnp.concatenate([eye, zz, zz, zz], axis=1)
    p2 = jnp.concatenate([zz, eye, zz, zz], axis=1)
    p3 = jnp.concatenate([jnp.zeros((128, 128), f32),
                          jnp.eye(128, dtype=f32)], axis=1)
    ymx, ymn, sy, sy2 = _feat(x1, x2, x3, p1, p2, p3, W4, b4[None])
    nf = jnp.float32(N)
    m4 = jnp.sum(sy[:, 0, :], axis=0) / nf
    v4 = jnp.sum(sy2[:, 0, :], axis=0) / nf - m4 * m4
    st4 = (jnp.zeros((8, m4.shape[0]), f32)
           .at[0].set(m4).at[1].set(v4).at[2].set(g4).at[3].set(be4))

    w7p = jnp.pad(W7, ((0, 0), (0, 118)))
    b7p = jnp.pad(b7, (0, 118))[None]
    out = _head(ymx[:, 0, :], ymn[:, 0, :], st4, W5, b5[None], g5[None],
                be5[None], W6, b6[None], g6[None], be6[None], w7p, b7p)
    return out[:, :10]


def kernel(pos, batch, W1, b1, g1, be1, W2, b2, g2, be2, W3, b3, g3, be3,
           W4, b4, g4, be4, W5, b5, g5, be5, W6, b6, g6, be6, W7, b7):
    return _forward(pos, W1, b1, g1, be1, W2, b2, g2, be2, W3, b3, g3, be3,
                    W4, b4, g4, be4, W5, b5, g5, be5, W6, b6, g6, be6,
                    W7, b7, B=16)
